# baseline (speedup 1.0000x reference)
import functools

import jax
import jax.numpy as jnp
from jax import lax
from jax.experimental import pallas as pl
from jax.experimental.pallas import tpu as pltpu
from jax.experimental.pallas import tpu_sc as plsc

_GAMMA = 12.0
_B = 1024
_NEG = 200
_NEG_PAD = 208
_D = 128
_NE = 100000
_NR = 1000
_NC = 2
_NS = 16
_NW = _NC * _NS
_SPLIT = 384
_BPW = _SPLIT // _NW
_NGROUP = _NEG_PAD // 16
_NGRP8 = _NEG // 8


def _sc_score(head_idx, rel_idx, neg_idx, entity_embedding,
              relation_embedding):
    mesh = plsc.VectorSubcoreMesh(core_axis_name="c", subcore_axis_name="s")

    @functools.partial(
        pl.kernel,
        out_type=jax.ShapeDtypeStruct((_SPLIT, _NEG_PAD), jnp.float32),
        mesh=mesh,
        scratch_types=[
            pltpu.VMEM((_BPW,), jnp.int32),
            pltpu.VMEM((_BPW,), jnp.int32),
            pltpu.VMEM((_BPW, _D), jnp.float32),
            pltpu.VMEM((_BPW, _D), jnp.float32),
            pltpu.VMEM((_BPW, 2, _NEG_PAD // 2), jnp.int32),
            pltpu.VMEM((2, _NEG_PAD, _D), jnp.float32),
            pltpu.VMEM((_NEG_PAD,), jnp.float32),
            pltpu.SemaphoreType.DMA,
            pltpu.SemaphoreType.DMA,
        ],
    )
    def k(head_idx_hbm, rel_idx_hbm, neg_hbm, ent_hbm, rel_emb_hbm, out_hbm,
          hidx_v, ridx_v, head_rows, rel_rows, neg_v, tails, score_row,
          sem0, sem1):
        wid = lax.axis_index("s") * _NC + lax.axis_index("c")
        base = wid * _BPW

        pltpu.sync_copy(head_idx_hbm.at[wid], hidx_v)
        pltpu.sync_copy(rel_idx_hbm.at[wid], ridx_v)
        pltpu.sync_copy(neg_hbm.at[pl.ds(base, _BPW)], neg_v)
        pltpu.async_copy(ent_hbm.at[hidx_v], head_rows, sem0).wait()
        pltpu.async_copy(rel_emb_hbm.at[ridx_v], rel_rows, sem0).wait()

        half = _NEG_PAD // 2
        iota16 = lax.iota(jnp.int32, 16)
        perms = [iota16 ^ k for k in (1, 2, 4, 8)]
        lane_eq = [iota16 == n for n in range(16)]

        def issue(b, buf, sem):
            pltpu.async_copy(
                ent_hbm.at[neg_v.at[b, 0]], tails.at[buf, pl.ds(0, half)],
                sem)
            pltpu.async_copy(
                ent_hbm.at[neg_v.at[b, 1]],
                tails.at[buf, pl.ds(half, half)], sem)

        def drain(buf, sem):
            dummy = ent_hbm.at[pl.ds(0, half)]
            pltpu.make_async_copy(dummy, tails.at[buf, pl.ds(0, half)],
                                  sem).wait()
            pltpu.make_async_copy(dummy, tails.at[buf, pl.ds(half, half)],
                                  sem).wait()

        def compute(b, tails_buf):
            qs = [head_rows[b, pl.ds(c * 16, 16)]
                  + rel_rows[b, pl.ds(c * 16, 16)] for c in range(8)]

            def body_g(g, carry2):
                vec = jnp.zeros((16,), jnp.float32)
                for n in range(16):
                    row = g * 16 + n
                    acc = jnp.abs(qs[0] - tails_buf[row, pl.ds(0, 16)])
                    for c in range(1, 8):
                        acc = acc + jnp.abs(
                            qs[c] - tails_buf[row, pl.ds(c * 16, 16)])
                    for p in perms:
                        acc = acc + acc.at[p].get(mode="promise_in_bounds")
                    vec = jnp.where(lane_eq[n], acc, vec)
                score_row[pl.ds(g * 16, 16)] = _GAMMA - vec
                return carry2

            lax.fori_loop(0, _NGROUP, body_g, 0)
            pltpu.sync_copy(score_row, out_hbm.at[base + b])

        issue(0, 0, sem0)

        def step(s, carry):
            issue(2 * s + 1, 1, sem1)
            drain(0, sem0)
            compute(2 * s, tails.at[0])

            @pl.when(s < _BPW // 2 - 1)
            def _():
                issue(2 * s + 2, 0, sem0)

            drain(1, sem1)
            compute(2 * s + 1, tails.at[1])
            return carry

        lax.fori_loop(0, _BPW // 2, step, 0)

    return k(head_idx, rel_idx, neg_idx, entity_embedding, relation_embedding)


def _tc_score(head_idx, rel_idx, neg_idx, entity_embedding,
              relation_embedding, nb, chunk=128):
    grid = (nb // chunk,)

    def body(hidx_s, ridx_s, neg_s, ent_any, relt_any, out_v, table_v, rel_v,
             q_v, sem0, sem1):
        i = pl.program_id(0)

        @pl.when(i == 0)
        def _():
            cp0 = pltpu.make_async_copy(ent_any, table_v, sem0)
            cp1 = pltpu.make_async_copy(relt_any, rel_v, sem1)
            cp0.start()
            cp1.start()
            cp0.wait()
            cp1.wait()

        ones = jnp.ones((_D, 1), jnp.float32)
        lane = jax.lax.broadcasted_iota(jnp.int32, (8, chunk), 1)

        def body_q(j, carry):
            h = hidx_s[0, 0, j]
            r = ridx_s[0, 0, j]
            q_v[pl.ds(j, 1), :] = (table_v[pl.ds(h, 1), :]
                                   + rel_v[pl.ds(r, 1), :])
            return carry

        lax.fori_loop(0, chunk, body_q, 0)

        def body_j(jj, accs):
            out_accs = list(accs)
            for u in range(8):
                j = 8 * jj + u
                q = q_v[pl.ds(j, 1), :]
                is_j = lane == j
                for g in range(_NGRP8):
                    rows = [table_v[pl.ds(neg_s[j, g * 8 + k], 1), :]
                            for k in range(8)]
                    d = jnp.abs(q - jnp.concatenate(rows, axis=0))
                    s = jax.lax.dot_general(
                        d, ones, (((1,), (0,)), ((), ())),
                        preferred_element_type=jnp.float32)
                    s_b = jnp.broadcast_to(s, (8, chunk))
                    out_accs[g] = jnp.where(is_j, _GAMMA - s_b, out_accs[g])
            return tuple(out_accs)

        init = tuple(
            jnp.zeros((8, chunk), jnp.float32) for _ in range(_NGRP8))
        accs = lax.fori_loop(0, chunk // 8, body_j, init)
        for g in range(_NGRP8):
            out_v[pl.ds(g * 8, 8), :] = accs[g]

    return pl.pallas_call(
        body,
        grid=grid,
        in_specs=[
            pl.BlockSpec((1, 1, chunk), lambda i: (i, 0, 0),
                         memory_space=pltpu.SMEM),
            pl.BlockSpec((1, 1, chunk), lambda i: (i, 0, 0),
                         memory_space=pltpu.SMEM),
            pl.BlockSpec((chunk, _NEG), lambda i: (i, 0),
                         memory_space=pltpu.SMEM),
            pl.BlockSpec(memory_space=pl.ANY),
            pl.BlockSpec(memory_space=pl.ANY),
        ],
        out_specs=pl.BlockSpec((_NEG, chunk), lambda i: (0, i)),
        out_shape=jax.ShapeDtypeStruct((_NEG, nb), jnp.float32),
        scratch_shapes=[
            pltpu.VMEM((_NE, _D), jnp.float32),
            pltpu.VMEM((_NR, _D), jnp.float32),
            pltpu.VMEM((chunk, _D), jnp.float32),
            pltpu.SemaphoreType.DMA,
            pltpu.SemaphoreType.DMA,
        ],
        compiler_params=pltpu.CompilerParams(
            vmem_limit_bytes=100 * 1024 * 1024),
    )(head_idx.reshape(-1, 1, chunk), rel_idx.reshape(-1, 1, chunk), neg_idx,
      entity_embedding, relation_embedding)


def kernel(positive_sample, negative_sample, entity_embedding,
           relation_embedding, entity_cov, relation_cov):
    del entity_cov, relation_cov
    head_idx = positive_sample[:, 0].astype(jnp.int32)
    rel_idx = positive_sample[:, 1].astype(jnp.int32)
    neg = negative_sample.astype(jnp.int32)

    neg_pad = jnp.concatenate(
        [neg[:_SPLIT], jnp.zeros((_SPLIT, _NEG_PAD - _NEG), jnp.int32)],
        axis=1)
    sc_out = _sc_score(head_idx[:_SPLIT].reshape(_NW, _BPW),
                       rel_idx[:_SPLIT].reshape(_NW, _BPW),
                       neg_pad.reshape(_SPLIT, 2, _NEG_PAD // 2),
                       entity_embedding, relation_embedding)
    tc_out = _tc_score(head_idx[_SPLIT:], rel_idx[_SPLIT:], neg[_SPLIT:],
                       entity_embedding, relation_embedding,
                       nb=_B - _SPLIT)
    return jnp.concatenate([sc_out[:, :_NEG], tc_out.T], axis=0)


# speedup vs baseline: 1.1187x; 1.1187x over previous
import functools

import jax
import jax.numpy as jnp
from jax import lax
from jax.experimental import pallas as pl
from jax.experimental.pallas import tpu as pltpu
from jax.experimental.pallas import tpu_sc as plsc

_GAMMA = 12.0
_B = 1024
_NEG = 200
_NEG_PAD = 208
_D = 128
_NE = 100000
_NR = 1000
_NC = 2
_NS = 16
_NW = _NC * _NS
_SPLIT = 512
_BPW = _SPLIT // _NW
_NGROUP = _NEG_PAD // 16
_NGRP8 = _NEG // 8


def _sc_score(head_idx, rel_idx, neg_idx, entity_embedding,
              relation_embedding):
    mesh = plsc.VectorSubcoreMesh(core_axis_name="c", subcore_axis_name="s")

    @functools.partial(
        pl.kernel,
        out_type=jax.ShapeDtypeStruct((_SPLIT, _NEG_PAD), jnp.float32),
        mesh=mesh,
        scratch_types=[
            pltpu.VMEM((_BPW,), jnp.int32),
            pltpu.VMEM((_BPW,), jnp.int32),
            pltpu.VMEM((_BPW, _D), jnp.float32),
            pltpu.VMEM((_BPW, _D), jnp.float32),
            pltpu.VMEM((_BPW, 2, _NEG_PAD // 2), jnp.int32),
            pltpu.VMEM((2, _NEG_PAD, _D), jnp.float32),
            pltpu.VMEM((_NEG_PAD,), jnp.float32),
            pltpu.SemaphoreType.DMA,
            pltpu.SemaphoreType.DMA,
        ],
    )
    def k(head_idx_hbm, rel_idx_hbm, neg_hbm, ent_hbm, rel_emb_hbm, out_hbm,
          hidx_v, ridx_v, head_rows, rel_rows, neg_v, tails, score_row,
          sem0, sem1):
        wid = lax.axis_index("s") * _NC + lax.axis_index("c")
        base = wid * _BPW

        pltpu.sync_copy(head_idx_hbm.at[wid], hidx_v)
        pltpu.sync_copy(rel_idx_hbm.at[wid], ridx_v)
        pltpu.sync_copy(neg_hbm.at[pl.ds(base, _BPW)], neg_v)
        pltpu.async_copy(ent_hbm.at[hidx_v], head_rows, sem0).wait()
        pltpu.async_copy(rel_emb_hbm.at[ridx_v], rel_rows, sem0).wait()

        half = _NEG_PAD // 2
        iota16 = lax.iota(jnp.int32, 16)
        perms = [iota16 ^ k for k in (1, 2, 4, 8)]
        lane_eq = [iota16 == n for n in range(16)]

        def issue(b, buf, sem):
            pltpu.async_copy(
                ent_hbm.at[neg_v.at[b, 0]], tails.at[buf, pl.ds(0, half)],
                sem)
            pltpu.async_copy(
                ent_hbm.at[neg_v.at[b, 1]],
                tails.at[buf, pl.ds(half, half)], sem)

        def drain(buf, sem):
            dummy = ent_hbm.at[pl.ds(0, half)]
            pltpu.make_async_copy(dummy, tails.at[buf, pl.ds(0, half)],
                                  sem).wait()
            pltpu.make_async_copy(dummy, tails.at[buf, pl.ds(half, half)],
                                  sem).wait()

        def compute(b, tails_buf):
            qs = [head_rows[b, pl.ds(c * 16, 16)]
                  + rel_rows[b, pl.ds(c * 16, 16)] for c in range(8)]

            def body_g(g, carry2):
                vec = jnp.zeros((16,), jnp.float32)
                for n in range(16):
                    row = g * 16 + n
                    acc = jnp.abs(qs[0] - tails_buf[row, pl.ds(0, 16)])
                    for c in range(1, 8):
                        acc = acc + jnp.abs(
                            qs[c] - tails_buf[row, pl.ds(c * 16, 16)])
                    for p in perms:
                        acc = acc + acc.at[p].get(mode="promise_in_bounds")
                    vec = jnp.where(lane_eq[n], acc, vec)
                score_row[pl.ds(g * 16, 16)] = _GAMMA - vec
                return carry2

            lax.fori_loop(0, _NGROUP, body_g, 0)
            pltpu.sync_copy(score_row, out_hbm.at[base + b])

        issue(0, 0, sem0)

        def step(s, carry):
            issue(2 * s + 1, 1, sem1)
            drain(0, sem0)
            compute(2 * s, tails.at[0])

            @pl.when(s < _BPW // 2 - 1)
            def _():
                issue(2 * s + 2, 0, sem0)

            drain(1, sem1)
            compute(2 * s + 1, tails.at[1])
            return carry

        lax.fori_loop(0, _BPW // 2, step, 0)

    return k(head_idx, rel_idx, neg_idx, entity_embedding, relation_embedding)


def _tc_score(head_idx, rel_idx, neg_idx, entity_embedding,
              relation_embedding, nb, chunk=128):
    grid = (nb // chunk,)

    def body(hidx_s, ridx_s, neg_s, ent_any, relt_any, out_v, table_v, rel_v,
             q_v, sem0, sem1):
        i = pl.program_id(0)

        @pl.when(i == 0)
        def _():
            cp0 = pltpu.make_async_copy(ent_any, table_v, sem0)
            cp1 = pltpu.make_async_copy(relt_any, rel_v, sem1)
            cp0.start()
            cp1.start()
            cp0.wait()
            cp1.wait()

        ones = jnp.ones((_D, 1), jnp.float32)
        lane = jax.lax.broadcasted_iota(jnp.int32, (8, chunk), 1)

        def body_q(j, carry):
            h = hidx_s[0, 0, j]
            r = ridx_s[0, 0, j]
            q_v[pl.ds(j, 1), :] = (table_v[pl.ds(h, 1), :]
                                   + rel_v[pl.ds(r, 1), :])
            return carry

        lax.fori_loop(0, chunk, body_q, 0)

        def body_j(jj, accs):
            out_accs = list(accs)
            for u in range(8):
                j = 8 * jj + u
                q = q_v[pl.ds(j, 1), :]
                is_j = lane == j
                for g in range(_NGRP8):
                    rows = [table_v[pl.ds(neg_s[j, g * 8 + k], 1), :]
                            for k in range(8)]
                    d = jnp.abs(q - jnp.concatenate(rows, axis=0))
                    s = jax.lax.dot_general(
                        d, ones, (((1,), (0,)), ((), ())),
                        preferred_element_type=jnp.float32)
                    s_b = jnp.broadcast_to(s, (8, chunk))
                    out_accs[g] = jnp.where(is_j, _GAMMA - s_b, out_accs[g])
            return tuple(out_accs)

        init = tuple(
            jnp.zeros((8, chunk), jnp.float32) for _ in range(_NGRP8))
        accs = lax.fori_loop(0, chunk // 8, body_j, init)
        for g in range(_NGRP8):
            out_v[pl.ds(g * 8, 8), :] = accs[g]

    return pl.pallas_call(
        body,
        grid=grid,
        in_specs=[
            pl.BlockSpec((1, 1, chunk), lambda i: (i, 0, 0),
                         memory_space=pltpu.SMEM),
            pl.BlockSpec((1, 1, chunk), lambda i: (i, 0, 0),
                         memory_space=pltpu.SMEM),
            pl.BlockSpec((chunk, _NEG), lambda i: (i, 0),
                         memory_space=pltpu.SMEM),
            pl.BlockSpec(memory_space=pl.ANY),
            pl.BlockSpec(memory_space=pl.ANY),
        ],
        out_specs=pl.BlockSpec((_NEG, chunk), lambda i: (0, i)),
        out_shape=jax.ShapeDtypeStruct((_NEG, nb), jnp.float32),
        scratch_shapes=[
            pltpu.VMEM((_NE, _D), jnp.float32),
            pltpu.VMEM((_NR, _D), jnp.float32),
            pltpu.VMEM((chunk, _D), jnp.float32),
            pltpu.SemaphoreType.DMA,
            pltpu.SemaphoreType.DMA,
        ],
        compiler_params=pltpu.CompilerParams(
            vmem_limit_bytes=100 * 1024 * 1024),
    )(head_idx.reshape(-1, 1, chunk), rel_idx.reshape(-1, 1, chunk), neg_idx,
      entity_embedding, relation_embedding)


def kernel(positive_sample, negative_sample, entity_embedding,
           relation_embedding, entity_cov, relation_cov):
    del entity_cov, relation_cov
    head_idx = positive_sample[:, 0].astype(jnp.int32)
    rel_idx = positive_sample[:, 1].astype(jnp.int32)
    neg = negative_sample.astype(jnp.int32)

    neg_pad = jnp.concatenate(
        [neg[:_SPLIT], jnp.zeros((_SPLIT, _NEG_PAD - _NEG), jnp.int32)],
        axis=1)
    sc_out = _sc_score(head_idx[:_SPLIT].reshape(_NW, _BPW),
                       rel_idx[:_SPLIT].reshape(_NW, _BPW),
                       neg_pad.reshape(_SPLIT, 2, _NEG_PAD // 2),
                       entity_embedding, relation_embedding)
    tc_out = _tc_score(head_idx[_SPLIT:], rel_idx[_SPLIT:], neg[_SPLIT:],
                       entity_embedding, relation_embedding,
                       nb=_B - _SPLIT)
    return jnp.concatenate([sc_out[:, :_NEG], tc_out.T], axis=0)
